# Initial kernel scaffold; baseline (speedup 1.0000x reference)
import jax
import jax.numpy as jnp
from jax.experimental import pallas as pl


def kernel(position_ids, table):
    raise NotImplementedError("write your pallas kernel here")


# speedup vs baseline: 2.2776x; 2.2776x over previous
import functools

import jax
import jax.numpy as jnp
from jax import lax
from jax.experimental import pallas as pl
from jax.experimental.pallas import tpu as pltpu
from jax.experimental.pallas import tpu_sc as plsc

_HIDDEN = 768
_CHUNK = 64


@functools.lru_cache(maxsize=None)
def _make_gather(n_ids: int, vocab: int, hidden: int):
    info = plsc.get_sparse_core_info()
    nw = info.num_cores * info.num_subcores
    assert n_ids % (8 * nw) == 0
    per_w = n_ids // nw
    chunk = min(_CHUNK, per_w)
    n_chunks = per_w // chunk
    assert per_w % chunk == 0

    mesh = plsc.VectorSubcoreMesh(core_axis_name="c", subcore_axis_name="s")

    @functools.partial(
        pl.kernel,
        mesh=mesh,
        out_type=jax.ShapeDtypeStruct((n_ids, hidden), jnp.float32),
        scratch_types=[
            pltpu.VMEM((per_w,), jnp.int32),
            pltpu.VMEM((chunk, hidden), jnp.float32),
            pltpu.SemaphoreType.DMA,
        ],
    )
    def gather_kernel(idx_hbm, table_hbm, out_hbm, idx_v, buf, sem):
        wid = lax.axis_index("s") * info.num_cores + lax.axis_index("c")
        base = wid * per_w
        pltpu.sync_copy(idx_hbm.at[pl.ds(base, per_w)], idx_v)

        def body(i, carry):
            off = i * chunk
            pltpu.async_copy(
                table_hbm.at[idx_v.at[pl.ds(off, chunk)]], buf, sem
            ).wait()
            pltpu.sync_copy(buf, out_hbm.at[pl.ds(base + off, chunk)])
            return carry

        lax.fori_loop(0, n_chunks, body, 0)

    return gather_kernel


def kernel(position_ids, table):
    batch, seq = position_ids.shape
    vocab, hidden = table.shape
    ids = position_ids.reshape(-1).astype(jnp.int32)
    out = _make_gather(ids.shape[0], vocab, hidden)(ids, table)
    return out.reshape(batch, seq, hidden)


# speedup vs baseline: 2.5317x; 1.1115x over previous
import functools

import jax
import jax.numpy as jnp
from jax import lax
from jax.experimental import pallas as pl
from jax.experimental.pallas import tpu as pltpu
from jax.experimental.pallas import tpu_sc as plsc

_HIDDEN = 768
_CHUNK = 64


@functools.lru_cache(maxsize=None)
def _make_gather(n_ids: int, vocab: int, hidden: int):
    info = plsc.get_sparse_core_info()
    nw = info.num_cores * info.num_subcores
    assert n_ids % (8 * nw) == 0
    per_w = n_ids // nw
    chunk = min(_CHUNK, per_w)
    n_chunks = per_w // chunk
    assert per_w % chunk == 0

    mesh = plsc.VectorSubcoreMesh(core_axis_name="c", subcore_axis_name="s")

    assert n_chunks >= 4 and n_chunks % 2 == 0

    @functools.partial(
        pl.kernel,
        mesh=mesh,
        out_type=jax.ShapeDtypeStruct((n_ids, hidden), jnp.float32),
        scratch_types=[
            pltpu.VMEM((per_w,), jnp.int32),
            pltpu.VMEM((chunk, hidden), jnp.float32),
            pltpu.VMEM((chunk, hidden), jnp.float32),
            pltpu.SemaphoreType.DMA,
            pltpu.SemaphoreType.DMA,
            pltpu.SemaphoreType.DMA,
            pltpu.SemaphoreType.DMA,
        ],
    )
    def gather_kernel(idx_hbm, table_hbm, out_hbm, idx_v, buf0, buf1,
                      sg0, sg1, so0, so1):
        wid = lax.axis_index("s") * info.num_cores + lax.axis_index("c")
        base = wid * per_w
        pltpu.sync_copy(idx_hbm.at[pl.ds(base, per_w)], idx_v)

        def g_copy(buf, sem, c):
            return pltpu.make_async_copy(
                table_hbm.at[idx_v.at[pl.ds(c * chunk, chunk)]], buf, sem)

        def o_copy(buf, sem, c):
            return pltpu.make_async_copy(
                buf, out_hbm.at[pl.ds(base + c * chunk, chunk)], sem)

        g_copy(buf0, sg0, 0).start()
        g_copy(buf1, sg1, 1).start()
        g_copy(buf0, sg0, 0).wait()
        o_copy(buf0, so0, 0).start()

        def body(k, carry):
            o_copy(buf0, so0, 2 * k).wait()
            g_copy(buf0, sg0, 2 * k + 2).start()
            g_copy(buf1, sg1, 2 * k + 1).wait()
            o_copy(buf1, so1, 2 * k + 1).start()
            o_copy(buf1, so1, 2 * k + 1).wait()
            g_copy(buf1, sg1, 2 * k + 3).start()
            g_copy(buf0, sg0, 2 * k + 2).wait()
            o_copy(buf0, so0, 2 * k + 2).start()
            return carry

        lax.fori_loop(0, (n_chunks - 2) // 2, body, 0)

        g_copy(buf1, sg1, n_chunks - 1).wait()
        o_copy(buf0, so0, n_chunks - 2).wait()
        o_copy(buf1, so1, n_chunks - 1).start()
        o_copy(buf1, so1, n_chunks - 1).wait()

    return gather_kernel


def kernel(position_ids, table):
    batch, seq = position_ids.shape
    vocab, hidden = table.shape
    ids = position_ids.reshape(-1).astype(jnp.int32)
    out = _make_gather(ids.shape[0], vocab, hidden)(ids, table)
    return out.reshape(batch, seq, hidden)
